# baseline (speedup 1.0000x reference)
import jax
import jax.numpy as jnp
from jax.experimental import pallas as pl
from jax.experimental.pallas import tpu as pltpu
from jax.experimental.pallas import tpu_sc as plsc

DIM = 64
CODEBOOK_SIZE = 1024
B = 16
N = 1024
BN = B * N
GATHER_WINDOW = 128


def _vq_kernel(xb_ref, x2_ref, eb_ref, e2_ref, dist_ref, ind_ref):
    xb = xb_ref[0]
    x2 = jnp.transpose(x2_ref[0])
    eb = eb_ref[...]
    e2 = e2_ref[...]
    xy = jax.lax.dot_general(
        xb, eb, (((1,), (1,)), ((), ())),
        preferred_element_type=jnp.float32,
    ) * -2.0
    d2 = (x2 + e2) + xy
    s = jnp.sqrt(jnp.maximum(d2, 0.0))
    dist_ref[0, 0] = -s
    mins = jnp.min(s, axis=1, keepdims=True)
    iota = jax.lax.broadcasted_iota(jnp.int32, (N, CODEBOOK_SIZE), 1)
    ind = jnp.min(jnp.where(s <= mins, iota, CODEBOOK_SIZE), axis=1,
                  keepdims=True)
    ind_ref[0] = jnp.transpose(ind)


def _tc_dist_argmin(xb, x2, eb, e2):
    return pl.pallas_call(
        _vq_kernel,
        grid=(B,),
        in_specs=[
            pl.BlockSpec((1, N, DIM), lambda i: (i, 0, 0)),
            pl.BlockSpec((1, 1, N), lambda i: (i, 0, 0)),
            pl.BlockSpec((CODEBOOK_SIZE, DIM), lambda i: (0, 0)),
            pl.BlockSpec((1, CODEBOOK_SIZE), lambda i: (0, 0)),
        ],
        out_specs=[
            pl.BlockSpec((1, 1, N, CODEBOOK_SIZE), lambda i: (0, i, 0, 0)),
            pl.BlockSpec((1, 1, N), lambda i: (i, 0, 0)),
        ],
        out_shape=[
            jax.ShapeDtypeStruct((1, B, N, CODEBOOK_SIZE), jnp.float32),
            jax.ShapeDtypeStruct((B, 1, N), jnp.int32),
        ],
        compiler_params=pltpu.CompilerParams(
            dimension_semantics=("arbitrary",)),
    )(xb, x2, eb, e2)


def _sc_gather(e_padded, ind_row):
    mesh = plsc.VectorSubcoreMesh(core_axis_name="core",
                                  subcore_axis_name="subcore")

    @pl.kernel(out_type=jax.ShapeDtypeStruct((BN, 128), jnp.float32),
               mesh=mesh)
    def gather_kernel(e_hbm, i_hbm, o_hbm):
        def body(i_vmem, o_vmem):
            pltpu.sync_copy(e_hbm.at[i_vmem.at[0]], o_vmem)

        pltpu.emit_pipeline(
            body,
            grid=(BN // GATHER_WINDOW,),
            in_specs=[pl.BlockSpec((1, GATHER_WINDOW),
                                   index_map=lambda i: (0, i))],
            out_specs=[pl.BlockSpec((GATHER_WINDOW, 128),
                                    index_map=lambda i: (i, 0))],
            core_axis_name=("core", "subcore"),
            dimension_semantics=(pltpu.PARALLEL,),
        )(i_hbm, o_hbm)

    return gather_kernel(e_padded, ind_row)


def kernel(x, embed):
    e = embed[0]
    eb = e.astype(jnp.bfloat16)
    e2 = jnp.sum(e ** 2, axis=-1)[None, :]
    xb = x.astype(jnp.bfloat16)
    x2 = jnp.sum(x ** 2, axis=-1)[:, None, :]
    dist, ind = _tc_dist_argmin(xb, x2, eb, e2)
    e_padded = jnp.pad(e, ((0, 0), (0, 128 - DIM)))
    q = _sc_gather(e_padded, ind.reshape(1, BN))
    return (q[:, :DIM].reshape(B, N, DIM), ind[:, 0, :], dist)


# speedup vs baseline: 1.2452x; 1.2452x over previous
import jax
import jax.numpy as jnp
from jax.experimental import pallas as pl
from jax.experimental.pallas import tpu as pltpu

DIM = 64
CODEBOOK_SIZE = 1024
B = 16
N = 1024


def _vq_kernel(xb_ref, x2_ref, eb_ref, e2_ref, dist_ref, q_ref, ind_ref):
    xb = xb_ref[0]
    x2 = jnp.transpose(x2_ref[0])
    eb = eb_ref[...]
    e2 = e2_ref[...]
    xy = jax.lax.dot_general(
        xb, eb, (((1,), (1,)), ((), ())),
        preferred_element_type=jnp.float32,
    ) * -2.0
    d2 = (x2 + e2) + xy
    s = jnp.sqrt(jnp.maximum(d2, 0.0))
    dist_ref[0, 0] = -s
    mins = jnp.min(s, axis=1, keepdims=True)
    iota = jax.lax.broadcasted_iota(
        jnp.int32, (N, CODEBOOK_SIZE), 1).astype(jnp.float32)
    cand = jnp.where(s <= mins, iota, float(CODEBOOK_SIZE))
    indf = jnp.min(cand, axis=1, keepdims=True)
    ind_ref[0] = jnp.transpose(indf).astype(jnp.int32)
    oh = jnp.where(cand == indf, 1.0, 0.0)
    q_ref[0] = jax.lax.dot_general(
        oh, eb, (((1,), (0,)), ((), ())),
        preferred_element_type=jnp.float32,
    )


def kernel(x, embed):
    e = embed[0]
    eb = e.astype(jnp.bfloat16)
    e2 = jnp.sum(e ** 2, axis=-1)[None, :]
    xb = x.astype(jnp.bfloat16)
    x2 = jnp.sum(x ** 2, axis=-1)[:, None, :]
    dist, q, ind = pl.pallas_call(
        _vq_kernel,
        grid=(B,),
        in_specs=[
            pl.BlockSpec((1, N, DIM), lambda i: (i, 0, 0)),
            pl.BlockSpec((1, 1, N), lambda i: (i, 0, 0)),
            pl.BlockSpec((CODEBOOK_SIZE, DIM), lambda i: (0, 0)),
            pl.BlockSpec((1, CODEBOOK_SIZE), lambda i: (0, 0)),
        ],
        out_specs=[
            pl.BlockSpec((1, 1, N, CODEBOOK_SIZE), lambda i: (0, i, 0, 0)),
            pl.BlockSpec((1, N, DIM), lambda i: (i, 0, 0)),
            pl.BlockSpec((1, 1, N), lambda i: (i, 0, 0)),
        ],
        out_shape=[
            jax.ShapeDtypeStruct((1, B, N, CODEBOOK_SIZE), jnp.float32),
            jax.ShapeDtypeStruct((B, N, DIM), jnp.float32),
            jax.ShapeDtypeStruct((B, 1, N), jnp.int32),
        ],
        compiler_params=pltpu.CompilerParams(
            dimension_semantics=("arbitrary",)),
    )(xb, x2, eb, e2)
    return (q, ind[:, 0, :], dist)
